# baseline (speedup 1.0000x reference)
import functools
import math

import jax
import jax.numpy as jnp
from jax import lax
from jax.experimental import pallas as pl
from jax.experimental.pallas import tpu as pltpu
from jax.experimental.pallas import tpu_sc as plsc

D_MODEL = 1024
D_MEM = 512
N_HEADS = 16
HEAD_DIM = D_MODEL // N_HEADS
K_RET = 32
CAPACITY = 100000
NEG = -1.0e30
FILL = -3.0e38


KEY_BLK = 2048
N_BLK = (CAPACITY + KEY_BLK - 1) // KEY_BLK
N_PAD = N_BLK * KEY_BLK


def _score_body(x_ref, keys_ref, out_ref, q_acc):
    j = pl.program_id(0)

    @pl.when(j == 0)
    def _():
        q_acc[...] = jnp.sum(x_ref[...], axis=1)

    kb = keys_ref[...]
    s = lax.dot_general(q_acc[...], kb, (((1,), (1,)), ((), ())),
                        preferred_element_type=jnp.float32,
                        precision=lax.Precision.HIGHEST)
    n2 = jnp.sum(kb * kb, axis=1)
    r = lax.rsqrt(jnp.maximum(n2, 1e-24))
    col = j * KEY_BLK + lax.broadcasted_iota(jnp.int32, s.shape, 1)
    out_ref[...] = jnp.where(col < CAPACITY, s * r[None, :], NEG)


def _scores(x, mem_keys):
    return pl.pallas_call(
        _score_body,
        grid=(N_BLK,),
        in_specs=[
            pl.BlockSpec(x.shape, lambda j: (0, 0, 0)),
            pl.BlockSpec((KEY_BLK, D_MODEL), lambda j: (j, 0)),
        ],
        out_specs=pl.BlockSpec((x.shape[0], KEY_BLK), lambda j: (0, j)),
        out_shape=jax.ShapeDtypeStruct((x.shape[0], N_PAD), jnp.float32),
        scratch_shapes=[pltpu.VMEM((x.shape[0], D_MODEL), jnp.float32)],
    )(x, mem_keys)


L = 16
CHUNK = N_PAD // 16
NVREG = CHUNK // L


def _merge32(S0, I0, S1, I1, vd, vid):
    ta = S1 >= vd
    hi = jnp.where(ta, S1, vd)
    hi_i = jnp.where(ta, I1, vid)
    lo = jnp.where(ta, vd, S1)
    lo_i = jnp.where(ta, vid, I1)
    S1n, I1n = plsc.sort_key_val(hi, hi_i)
    lod, lod_i = plsc.sort_key_val(lo, lo_i, descending=True)
    tb = S0 >= lod
    hi2 = jnp.where(tb, S0, lod)
    hi2_i = jnp.where(tb, I0, lod_i)
    S0n, I0n = plsc.sort_key_val(hi2, hi2_i)
    return S0n, I0n, S1n, I1n


def _topk_init():
    return (jnp.full((L,), FILL, jnp.float32), jnp.zeros((L,), jnp.int32),
            jnp.full((L,), FILL, jnp.float32), jnp.zeros((L,), jnp.int32),
            jnp.float32(FILL))


def _topk_step(carry, v, vi):
    S0, I0, S1, I1, thr = carry
    vd, vid = plsc.sort_key_val(v, vi, descending=True)

    def do(_):
        S0n, I0n, S1n, I1n = _merge32(S0, I0, S1, I1, vd, vid)
        return (S0n, I0n, S1n, I1n, S0n[0])

    def skip(_):
        return (S0, I0, S1, I1, thr)

    return lax.cond(vd[0] > thr, do, skip, None)


def _topk_gather_body(sims_hbm, vals_hbm, out_hbm,
                      simbuf, stk, sti, shk, shi, ck, ci, rows, sem):
    b = lax.axis_index("c")
    s = lax.axis_index("s")
    base = (s * CHUNK).astype(jnp.int32)
    pltpu.sync_copy(sims_hbm.at[b, pl.ds(s * CHUNK, CHUNK)], simbuf)

    def body(j, carry):
        v = simbuf[pl.ds(j * L, L)]
        vi = base + j * L + lax.iota(jnp.int32, L)
        return _topk_step(carry, v, vi)

    S0, I0, S1, I1, _ = lax.fori_loop(0, NVREG, body, _topk_init())

    stk[pl.ds(0, L)] = S0
    stk[pl.ds(L, L)] = S1
    sti[pl.ds(0, L)] = I0
    sti[pl.ds(L, L)] = I1
    pltpu.sync_copy(stk, shk.at[pl.ds(s * 2 * L, 2 * L)])
    pltpu.sync_copy(sti, shi.at[pl.ds(s * 2 * L, 2 * L)])
    plsc.subcore_barrier()

    @pl.when(s == 0)
    def _():
        pltpu.sync_copy(shk, ck)
        pltpu.sync_copy(shi, ci)

        def body2(m, carry):
            v = ck[pl.ds(m * L, L)]
            vi = ci[pl.ds(m * L, L)]
            return _topk_step(carry, v, vi)

        _, I0f, _, I1f, _ = lax.fori_loop(0, 2 * L, body2, _topk_init())
        sti[pl.ds(0, L)] = I0f
        sti[pl.ds(L, L)] = I1f
        pltpu.sync_copy(sti, shi.at[pl.ds(0, 2 * L)])
        pltpu.async_copy(vals_hbm.at[sti], rows, sem).wait()
        pltpu.sync_copy(rows, out_hbm.at[b])


def _topk_gather(sims, mem_values):
    B = sims.shape[0]
    mesh = plsc.VectorSubcoreMesh(core_axis_name="c", subcore_axis_name="s")
    f = functools.partial(
        pl.kernel,
        out_type=jax.ShapeDtypeStruct((B, K_RET, D_MEM), jnp.float32),
        mesh=mesh,
        compiler_params=pltpu.CompilerParams(needs_layout_passes=False),
        scratch_types=[
            pltpu.VMEM((CHUNK,), jnp.float32),
            pltpu.VMEM((2 * L,), jnp.float32),
            pltpu.VMEM((2 * L,), jnp.int32),
            pltpu.VMEM_SHARED((16 * 2 * L,), jnp.float32),
            pltpu.VMEM_SHARED((16 * 2 * L,), jnp.int32),
            pltpu.VMEM((16 * 2 * L,), jnp.float32),
            pltpu.VMEM((16 * 2 * L,), jnp.int32),
            pltpu.VMEM((K_RET, D_MEM), jnp.float32),
            pltpu.SemaphoreType.DMA,
        ],
    )(_topk_gather_body)
    return f(sims, mem_values)


T_BLK = 512
SCALE = 1.0 / math.sqrt(HEAD_DIM)


def _attn_body(x_ref, retr_ref, wq_ref, wk_ref, wv_ref, wo_ref, g_ref,
               out_ref):
    xb = x_ref[0]
    g = jax.nn.sigmoid(g_ref[0, 0])
    q = lax.dot_general(xb, wq_ref[...], (((1,), (1,)), ((), ())),
                        preferred_element_type=jnp.float32)
    retr = retr_ref[0]
    kp = lax.dot_general(retr, wk_ref[...], (((1,), (1,)), ((), ())),
                         preferred_element_type=jnp.float32)
    v = lax.dot_general(retr, wv_ref[...], (((1,), (1,)), ((), ())),
                        preferred_element_type=jnp.float32)
    outs = []
    for h in range(N_HEADS):
        sl = slice(h * HEAD_DIM, (h + 1) * HEAD_DIM)
        sc = lax.dot_general(q[:, sl], kp[:, sl], (((1,), (1,)), ((), ())),
                             preferred_element_type=jnp.float32) * SCALE
        m = jnp.max(sc, axis=1, keepdims=True)
        p = jnp.exp(sc - m)
        pn = p / jnp.sum(p, axis=1, keepdims=True)
        outs.append(lax.dot_general(pn, v[:, sl], (((1,), (0,)), ((), ())),
                                    preferred_element_type=jnp.float32))
    ob = jnp.concatenate(outs, axis=1)
    ctx = lax.dot_general(ob, wo_ref[...], (((1,), (1,)), ((), ())),
                          preferred_element_type=jnp.float32)
    out_ref[0] = xb + g * ctx


def _attend(x, retrieved, Wq, Wk, Wv, Wo, gate_logit):
    B, T, _ = x.shape
    g2 = gate_logit.reshape(1, 1)
    return pl.pallas_call(
        _attn_body,
        grid=(B, T // T_BLK),
        in_specs=[
            pl.BlockSpec((1, T_BLK, D_MODEL), lambda b, t: (b, t, 0)),
            pl.BlockSpec((1, K_RET, D_MEM), lambda b, t: (b, 0, 0)),
            pl.BlockSpec((D_MODEL, D_MODEL), lambda b, t: (0, 0)),
            pl.BlockSpec((D_MODEL, D_MEM), lambda b, t: (0, 0)),
            pl.BlockSpec((D_MODEL, D_MEM), lambda b, t: (0, 0)),
            pl.BlockSpec((D_MODEL, D_MODEL), lambda b, t: (0, 0)),
            pl.BlockSpec((1, 1), lambda b, t: (0, 0)),
        ],
        out_specs=pl.BlockSpec((1, T_BLK, D_MODEL), lambda b, t: (b, t, 0)),
        out_shape=jax.ShapeDtypeStruct(x.shape, jnp.float32),
    )(x, retrieved, Wq, Wk, Wv, Wo, g2)


def kernel(x, mem_keys, mem_values, Wq, Wk, Wv, Wo, gate_logit):
    sims = _scores(x, mem_keys)
    retrieved = _topk_gather(sims, mem_values)
    return _attend(x, retrieved, Wq, Wk, Wv, Wo, gate_logit)


# speedup vs baseline: 1.6058x; 1.6058x over previous
import functools
import math

import jax
import jax.numpy as jnp
from jax import lax
from jax.experimental import pallas as pl
from jax.experimental.pallas import tpu as pltpu
from jax.experimental.pallas import tpu_sc as plsc

D_MODEL = 1024
D_MEM = 512
N_HEADS = 16
HEAD_DIM = D_MODEL // N_HEADS
K_RET = 32
CAPACITY = 100000
NEG = -1.0e30
FILL = -3.0e38


KEY_BLK = 2048
N_BLK = (CAPACITY + KEY_BLK - 1) // KEY_BLK
N_PAD = N_BLK * KEY_BLK


def _score_body(x_ref, keys_ref, out_ref, q_acc):
    j = pl.program_id(0)

    @pl.when(j == 0)
    def _():
        q_acc[...] = jnp.sum(x_ref[...], axis=1)

    kb = keys_ref[...]
    sT = lax.dot_general(kb, q_acc[...], (((1,), (1,)), ((), ())),
                         preferred_element_type=jnp.float32)
    acc = kb[:, 0:128] * kb[:, 0:128]
    for g in range(1, 8):
        c = kb[:, g * 128:(g + 1) * 128]
        acc = acc + c * c
    n2 = jnp.sum(acc, axis=1)
    r = lax.rsqrt(jnp.maximum(n2, 1e-24))
    res = sT * r[:, None]

    @pl.when(j < N_BLK - 1)
    def _():
        out_ref[...] = res

    @pl.when(j == N_BLK - 1)
    def _():
        row = j * KEY_BLK + lax.broadcasted_iota(jnp.int32, sT.shape, 0)
        out_ref[...] = jnp.where(row < CAPACITY, res, NEG)


def _scores(x, mem_keys):
    return pl.pallas_call(
        _score_body,
        grid=(N_BLK,),
        in_specs=[
            pl.BlockSpec(x.shape, lambda j: (0, 0, 0)),
            pl.BlockSpec((KEY_BLK, D_MODEL), lambda j: (j, 0)),
        ],
        out_specs=pl.BlockSpec((KEY_BLK, x.shape[0]), lambda j: (j, 0)),
        out_shape=jax.ShapeDtypeStruct((N_PAD, x.shape[0]), jnp.float32),
        scratch_shapes=[pltpu.VMEM((x.shape[0], D_MODEL), jnp.float32)],
    )(x, mem_keys)


L = 16
CHUNK = N_PAD // 16
NVREG = CHUNK // L


def _merge32(S0, I0, S1, I1, vd, vid):
    ta = S1 >= vd
    hi = jnp.where(ta, S1, vd)
    hi_i = jnp.where(ta, I1, vid)
    lo = jnp.where(ta, vd, S1)
    lo_i = jnp.where(ta, vid, I1)
    S1n, I1n = plsc.sort_key_val(hi, hi_i)
    lod, lod_i = plsc.sort_key_val(lo, lo_i, descending=True)
    tb = S0 >= lod
    hi2 = jnp.where(tb, S0, lod)
    hi2_i = jnp.where(tb, I0, lod_i)
    S0n, I0n = plsc.sort_key_val(hi2, hi2_i)
    return S0n, I0n, S1n, I1n


def _topk_init():
    return (jnp.full((L,), FILL, jnp.float32), jnp.zeros((L,), jnp.int32),
            jnp.full((L,), FILL, jnp.float32), jnp.zeros((L,), jnp.int32),
            jnp.float32(FILL))


def _topk_step(carry, v, vi):
    S0, I0, S1, I1, thr = carry
    vd, vid = plsc.sort_key_val(v, vi, descending=True)

    def do(_):
        S0n, I0n, S1n, I1n = _merge32(S0, I0, S1, I1, vd, vid)
        return (S0n, I0n, S1n, I1n, S0n[0])

    def skip(_):
        return (S0, I0, S1, I1, thr)

    return lax.cond(vd[0] > thr, do, skip, None)


def _topk_gather_body(sims_hbm, vals_hbm, out_hbm,
                      simbuf, stk, sti, shk, shi, ck, ci, rows, sem):
    b = lax.axis_index("c")
    s = lax.axis_index("s")
    base = (s * CHUNK).astype(jnp.int32)
    pltpu.sync_copy(sims_hbm.at[pl.ds(s * 2 * CHUNK, 2 * CHUNK)], simbuf)
    boff = jnp.full((L,), b, jnp.int32)

    def body(j, carry):
        ridx = j * L + lax.iota(jnp.int32, L)
        v = plsc.load_gather(simbuf, [2 * ridx + boff])
        vi = base + ridx
        return _topk_step(carry, v, vi)

    S0, I0, S1, I1, _ = lax.fori_loop(0, NVREG, body, _topk_init())

    stk[pl.ds(0, L)] = S0
    stk[pl.ds(L, L)] = S1
    sti[pl.ds(0, L)] = I0
    sti[pl.ds(L, L)] = I1
    pltpu.sync_copy(stk, shk.at[pl.ds(s * 2 * L, 2 * L)])
    pltpu.sync_copy(sti, shi.at[pl.ds(s * 2 * L, 2 * L)])
    plsc.subcore_barrier()

    @pl.when(s == 0)
    def _():
        pltpu.sync_copy(shk, ck)
        pltpu.sync_copy(shi, ci)

        def body2(m, carry):
            v = ck[pl.ds(m * L, L)]
            vi = ci[pl.ds(m * L, L)]
            return _topk_step(carry, v, vi)

        _, I0f, _, I1f, _ = lax.fori_loop(0, 2 * L, body2, _topk_init())
        sti[pl.ds(0, L)] = I0f
        sti[pl.ds(L, L)] = I1f
        pltpu.sync_copy(sti, shi.at[pl.ds(0, 2 * L)])
        pltpu.async_copy(vals_hbm.at[sti], rows, sem).wait()
        pltpu.sync_copy(rows, out_hbm.at[b])


def _topk_gather(sims_t, mem_values):
    B = sims_t.shape[1]
    sims_flat = sims_t.reshape(-1)
    mesh = plsc.VectorSubcoreMesh(core_axis_name="c", subcore_axis_name="s")
    f = functools.partial(
        pl.kernel,
        out_type=jax.ShapeDtypeStruct((B, K_RET, D_MEM), jnp.float32),
        mesh=mesh,
        compiler_params=pltpu.CompilerParams(needs_layout_passes=False),
        scratch_types=[
            pltpu.VMEM((2 * CHUNK,), jnp.float32),
            pltpu.VMEM((2 * L,), jnp.float32),
            pltpu.VMEM((2 * L,), jnp.int32),
            pltpu.VMEM_SHARED((16 * 2 * L,), jnp.float32),
            pltpu.VMEM_SHARED((16 * 2 * L,), jnp.int32),
            pltpu.VMEM((16 * 2 * L,), jnp.float32),
            pltpu.VMEM((16 * 2 * L,), jnp.int32),
            pltpu.VMEM((K_RET, D_MEM), jnp.float32),
            pltpu.SemaphoreType.DMA,
        ],
    )(_topk_gather_body)
    return f(sims_flat, mem_values)


T_BLK = 512
SCALE = 1.0 / math.sqrt(HEAD_DIM)


def _attn_body(x_ref, retr_ref, wq_ref, wk_ref, wv_ref, wo_ref, g_ref,
               out_ref):
    xb = x_ref[0]
    g = jax.nn.sigmoid(g_ref[0, 0])
    q = lax.dot_general(xb, wq_ref[...], (((1,), (1,)), ((), ())),
                        preferred_element_type=jnp.float32)
    retr = retr_ref[0]
    kp = lax.dot_general(retr, wk_ref[...], (((1,), (1,)), ((), ())),
                         preferred_element_type=jnp.float32)
    v = lax.dot_general(retr, wv_ref[...], (((1,), (1,)), ((), ())),
                        preferred_element_type=jnp.float32)
    outs = []
    for h in range(N_HEADS):
        sl = slice(h * HEAD_DIM, (h + 1) * HEAD_DIM)
        sc = lax.dot_general(q[:, sl], kp[:, sl], (((1,), (1,)), ((), ())),
                             preferred_element_type=jnp.float32) * SCALE
        m = jnp.max(sc, axis=1, keepdims=True)
        p = jnp.exp(sc - m)
        pn = p / jnp.sum(p, axis=1, keepdims=True)
        outs.append(lax.dot_general(pn, v[:, sl], (((1,), (0,)), ((), ())),
                                    preferred_element_type=jnp.float32))
    ob = jnp.concatenate(outs, axis=1)
    ctx = lax.dot_general(ob, wo_ref[...], (((1,), (1,)), ((), ())),
                          preferred_element_type=jnp.float32)
    out_ref[0] = xb + g * ctx


def _attend(x, retrieved, Wq, Wk, Wv, Wo, gate_logit):
    B, T, _ = x.shape
    g2 = gate_logit.reshape(1, 1)
    return pl.pallas_call(
        _attn_body,
        grid=(B, T // T_BLK),
        in_specs=[
            pl.BlockSpec((1, T_BLK, D_MODEL), lambda b, t: (b, t, 0)),
            pl.BlockSpec((1, K_RET, D_MEM), lambda b, t: (b, 0, 0)),
            pl.BlockSpec((D_MODEL, D_MODEL), lambda b, t: (0, 0)),
            pl.BlockSpec((D_MODEL, D_MEM), lambda b, t: (0, 0)),
            pl.BlockSpec((D_MODEL, D_MEM), lambda b, t: (0, 0)),
            pl.BlockSpec((D_MODEL, D_MODEL), lambda b, t: (0, 0)),
            pl.BlockSpec((1, 1), lambda b, t: (0, 0)),
        ],
        out_specs=pl.BlockSpec((1, T_BLK, D_MODEL), lambda b, t: (b, t, 0)),
        out_shape=jax.ShapeDtypeStruct(x.shape, jnp.float32),
    )(x, retrieved, Wq, Wk, Wv, Wo, g2)


def kernel(x, mem_keys, mem_values, Wq, Wk, Wv, Wo, gate_logit):
    sims = _scores(x, mem_keys)
    retrieved = _topk_gather(sims, mem_values)
    return _attend(x, retrieved, Wq, Wk, Wv, Wo, gate_logit)


# speedup vs baseline: 1.6488x; 1.0268x over previous
import functools
import math

import jax
import jax.numpy as jnp
from jax import lax
from jax.experimental import pallas as pl
from jax.experimental.pallas import tpu as pltpu
from jax.experimental.pallas import tpu_sc as plsc

D_MODEL = 1024
D_MEM = 512
N_HEADS = 16
HEAD_DIM = D_MODEL // N_HEADS
K_RET = 32
CAPACITY = 100000
NEG = -1.0e30
FILL = -3.0e38


KEY_BLK = 2048
N_BLK = (CAPACITY + KEY_BLK - 1) // KEY_BLK
N_PAD = N_BLK * KEY_BLK


def _score_body(x_ref, keys_ref, out_ref, q_acc):
    j = pl.program_id(0)

    @pl.when(j == 0)
    def _():
        q_acc[...] = jnp.sum(x_ref[...], axis=1)

    kb = keys_ref[...]
    sT = lax.dot_general(kb, q_acc[...], (((1,), (1,)), ((), ())),
                         preferred_element_type=jnp.float32)
    acc = kb[:, 0:128] * kb[:, 0:128]
    for g in range(1, 8):
        c = kb[:, g * 128:(g + 1) * 128]
        acc = acc + c * c
    n2 = jnp.sum(acc, axis=1)
    r = lax.rsqrt(jnp.maximum(n2, 1e-24))
    res = sT * r[:, None]

    @pl.when(j < N_BLK - 1)
    def _():
        out_ref[...] = res

    @pl.when(j == N_BLK - 1)
    def _():
        row = j * KEY_BLK + lax.broadcasted_iota(jnp.int32, sT.shape, 0)
        out_ref[...] = jnp.where(row < CAPACITY, res, NEG)


def _scores(x, mem_keys):
    return pl.pallas_call(
        _score_body,
        grid=(N_BLK,),
        in_specs=[
            pl.BlockSpec(x.shape, lambda j: (0, 0, 0)),
            pl.BlockSpec((KEY_BLK, D_MODEL), lambda j: (j, 0)),
        ],
        out_specs=pl.BlockSpec((KEY_BLK, x.shape[0]), lambda j: (j, 0)),
        out_shape=jax.ShapeDtypeStruct((N_PAD, x.shape[0]), jnp.float32),
        scratch_shapes=[pltpu.VMEM((x.shape[0], D_MODEL), jnp.float32)],
    )(x, mem_keys)


L = 16
CHUNK = N_PAD // 16
NVREG = CHUNK // L


def _merge32(S0, I0, S1, I1, vd, vid):
    ta = S1 >= vd
    hi = jnp.where(ta, S1, vd)
    hi_i = jnp.where(ta, I1, vid)
    lo = jnp.where(ta, vd, S1)
    lo_i = jnp.where(ta, vid, I1)
    S1n, I1n = plsc.sort_key_val(hi, hi_i)
    lod, lod_i = plsc.sort_key_val(lo, lo_i, descending=True)
    tb = S0 >= lod
    hi2 = jnp.where(tb, S0, lod)
    hi2_i = jnp.where(tb, I0, lod_i)
    S0n, I0n = plsc.sort_key_val(hi2, hi2_i)
    return S0n, I0n, S1n, I1n


def _topk_init():
    return (jnp.full((L,), FILL, jnp.float32), jnp.zeros((L,), jnp.int32),
            jnp.full((L,), FILL, jnp.float32), jnp.zeros((L,), jnp.int32),
            jnp.float32(FILL))


def _topk_step(carry, v, vi):
    S0, I0, S1, I1, thr = carry
    vd, vid = plsc.sort_key_val(v, vi, descending=True)

    def do(_):
        S0n, I0n, S1n, I1n = _merge32(S0, I0, S1, I1, vd, vid)
        return (S0n, I0n, S1n, I1n, S0n[0])

    def skip(_):
        return (S0, I0, S1, I1, thr)

    return lax.cond(vd[0] > thr, do, skip, None)


def _topk_gather_body(sims_hbm, vals_hbm, out_hbm,
                      simbuf, stk, sti, shk, shi, ck, ci, rows, sem):
    b = lax.axis_index("c")
    s = lax.axis_index("s")
    base = (s * CHUNK).astype(jnp.int32)
    pltpu.sync_copy(sims_hbm.at[pl.ds(s * 2 * CHUNK, 2 * CHUNK)], simbuf)
    boff = jnp.full((L,), b, jnp.int32)

    def body(j, carry):
        ridx = j * L + lax.iota(jnp.int32, L)
        v = plsc.load_gather(simbuf, [2 * ridx + boff])
        vi = base + ridx
        return _topk_step(carry, v, vi)

    S0, I0, S1, I1, _ = lax.fori_loop(0, NVREG, body, _topk_init())

    stk[pl.ds(0, L)] = S0
    stk[pl.ds(L, L)] = S1
    sti[pl.ds(0, L)] = I0
    sti[pl.ds(L, L)] = I1
    pltpu.sync_copy(stk, shk.at[pl.ds(s * 2 * L, 2 * L)])
    pltpu.sync_copy(sti, shi.at[pl.ds(s * 2 * L, 2 * L)])
    plsc.subcore_barrier()

    @pl.when(s == 0)
    def _():
        pltpu.sync_copy(shk, ck)
        pltpu.sync_copy(shi, ci)

        def body2(m, carry):
            v = ck[pl.ds(m * L, L)]
            vi = ci[pl.ds(m * L, L)]
            return _topk_step(carry, v, vi)

        _, I0f, _, I1f, _ = lax.fori_loop(0, 2 * L, body2, _topk_init())
        sti[pl.ds(0, L)] = I0f
        sti[pl.ds(L, L)] = I1f
        pltpu.sync_copy(sti, shi.at[pl.ds(0, 2 * L)])
        pltpu.async_copy(vals_hbm.at[sti], rows, sem).wait()
        pltpu.sync_copy(rows, out_hbm.at[b])


def _topk_gather(sims_t, mem_values):
    B = sims_t.shape[1]
    sims_flat = sims_t.reshape(-1)
    mesh = plsc.VectorSubcoreMesh(core_axis_name="c", subcore_axis_name="s")
    f = functools.partial(
        pl.kernel,
        out_type=jax.ShapeDtypeStruct((B, K_RET, D_MEM), jnp.float32),
        mesh=mesh,
        compiler_params=pltpu.CompilerParams(needs_layout_passes=False),
        scratch_types=[
            pltpu.VMEM((2 * CHUNK,), jnp.float32),
            pltpu.VMEM((2 * L,), jnp.float32),
            pltpu.VMEM((2 * L,), jnp.int32),
            pltpu.VMEM_SHARED((16 * 2 * L,), jnp.float32),
            pltpu.VMEM_SHARED((16 * 2 * L,), jnp.int32),
            pltpu.VMEM((16 * 2 * L,), jnp.float32),
            pltpu.VMEM((16 * 2 * L,), jnp.int32),
            pltpu.VMEM((K_RET, D_MEM), jnp.float32),
            pltpu.SemaphoreType.DMA,
        ],
    )(_topk_gather_body)
    return f(sims_flat, mem_values)


T_BLK = 512
SCALE = 1.0 / math.sqrt(HEAD_DIM)


def _attn_body(x_ref, retr_ref, wq_ref, wk_ref, wv_ref, wo_ref, g_ref,
               out_ref):
    xb = x_ref[0]
    g = jax.nn.sigmoid(g_ref[0, 0])
    xb16 = xb.astype(jnp.bfloat16)
    q = lax.dot_general(xb16, wq_ref[...].astype(jnp.bfloat16),
                        (((1,), (1,)), ((), ())),
                        preferred_element_type=jnp.float32)
    retr = retr_ref[0].astype(jnp.bfloat16)
    kp = lax.dot_general(retr, wk_ref[...].astype(jnp.bfloat16),
                         (((1,), (1,)), ((), ())),
                         preferred_element_type=jnp.float32)
    v = lax.dot_general(retr, wv_ref[...].astype(jnp.bfloat16),
                        (((1,), (1,)), ((), ())),
                        preferred_element_type=jnp.float32)
    outs = []
    for h in range(N_HEADS):
        sl = slice(h * HEAD_DIM, (h + 1) * HEAD_DIM)
        sc = lax.dot_general(q[:, sl].astype(jnp.bfloat16),
                             kp[:, sl].astype(jnp.bfloat16),
                             (((1,), (1,)), ((), ())),
                             preferred_element_type=jnp.float32) * SCALE
        m = jnp.max(sc, axis=1, keepdims=True)
        p = jnp.exp(sc - m)
        pn = p / jnp.sum(p, axis=1, keepdims=True)
        outs.append(lax.dot_general(pn.astype(jnp.bfloat16),
                                    v[:, sl].astype(jnp.bfloat16),
                                    (((1,), (0,)), ((), ())),
                                    preferred_element_type=jnp.float32))
    ob = jnp.concatenate(outs, axis=1)
    ctx = lax.dot_general(ob.astype(jnp.bfloat16),
                          wo_ref[...].astype(jnp.bfloat16),
                          (((1,), (1,)), ((), ())),
                          preferred_element_type=jnp.float32)
    out_ref[0] = xb + g * ctx


def _attend(x, retrieved, Wq, Wk, Wv, Wo, gate_logit):
    B, T, _ = x.shape
    g2 = gate_logit.reshape(1, 1)
    return pl.pallas_call(
        _attn_body,
        grid=(B, T // T_BLK),
        in_specs=[
            pl.BlockSpec((1, T_BLK, D_MODEL), lambda b, t: (b, t, 0)),
            pl.BlockSpec((1, K_RET, D_MEM), lambda b, t: (b, 0, 0)),
            pl.BlockSpec((D_MODEL, D_MODEL), lambda b, t: (0, 0)),
            pl.BlockSpec((D_MODEL, D_MEM), lambda b, t: (0, 0)),
            pl.BlockSpec((D_MODEL, D_MEM), lambda b, t: (0, 0)),
            pl.BlockSpec((D_MODEL, D_MODEL), lambda b, t: (0, 0)),
            pl.BlockSpec((1, 1), lambda b, t: (0, 0)),
        ],
        out_specs=pl.BlockSpec((1, T_BLK, D_MODEL), lambda b, t: (b, t, 0)),
        out_shape=jax.ShapeDtypeStruct(x.shape, jnp.float32),
    )(x, retrieved, Wq, Wk, Wv, Wo, g2)


def kernel(x, mem_keys, mem_values, Wq, Wk, Wv, Wo, gate_logit):
    sims = _scores(x, mem_keys)
    retrieved = _topk_gather(sims, mem_values)
    return _attend(x, retrieved, Wq, Wk, Wv, Wo, gate_logit)
